# baseline (speedup 1.0000x reference)
import functools

import jax
import jax.numpy as jnp
from jax import lax
from jax.experimental import pallas as pl
from jax.experimental.pallas import tpu as pltpu
from jax.experimental.pallas import tpu_sc as plsc

_NC = 2
_NS = 16
_NW = _NC * _NS
_CH = 128


def _sc_gather(table_flat, idx2d, n_total, d):
    n_per_w = n_total // _NW
    n_chunks = n_per_w // _CH
    mesh = plsc.VectorSubcoreMesh(core_axis_name="c", subcore_axis_name="s")

    @functools.partial(
        pl.kernel,
        mesh=mesh,
        out_type=jax.ShapeDtypeStruct((n_total, d), jnp.float32),
        scratch_types=[
            pltpu.VMEM((n_chunks, _CH), jnp.int32),
            pltpu.VMEM((n_per_w, d), jnp.float32),
            pltpu.SemaphoreType.DMA,
        ],
        compiler_params=pltpu.CompilerParams(use_tc_tiling_on_sc=False),
    )
    def gather_k(table_hbm, idx_hbm, out_hbm, idx_v, rows_v, sem):
        wid = lax.axis_index("s") * _NC + lax.axis_index("c")
        pltpu.sync_copy(idx_hbm.at[wid], idx_v)
        copies = []
        for j in range(n_chunks):
            copies.append(
                pltpu.async_copy(
                    table_hbm.at[idx_v.at[j]],
                    rows_v.at[pl.ds(j * _CH, _CH)],
                    sem,
                )
            )
        for c in copies:
            c.wait()
        pltpu.sync_copy(rows_v, out_hbm.at[pl.ds(wid * n_per_w, n_per_w)])

    return gather_k(table_flat, idx2d)


def _mlp_body(h_ref, g_ref, b_ref, w1_ref, b1_ref, w2_ref, b2_ref,
              w3_ref, b3_ref, o_ref):
    h = h_ref[...]
    mu = jnp.mean(h, axis=-1, keepdims=True)
    c = h - mu
    var = jnp.mean(c * c, axis=-1, keepdims=True)
    hn = c * lax.rsqrt(var + 1e-5) * g_ref[...] + b_ref[...]
    a = jnp.dot(hn, w1_ref[...], preferred_element_type=jnp.float32)
    a = jnp.maximum(a + b1_ref[...], 0.0)
    a = jnp.dot(a, w2_ref[...], preferred_element_type=jnp.float32)
    a = jnp.maximum(a + b2_ref[...], 0.0)
    o = jnp.dot(a, w3_ref[...], preferred_element_type=jnp.float32)
    o_ref[...] = o + b3_ref[...]


def _tc_mlp(h, ln_g, ln_b, w1, b1, w2, b2, w3, b3, blk):
    bsz, hdim = h.shape
    ncls = w3.shape[1]
    grid = bsz // blk
    full = lambda shape: pl.BlockSpec(shape, lambda i: (0,) * len(shape))
    return pl.pallas_call(
        _mlp_body,
        grid=(grid,),
        in_specs=[
            pl.BlockSpec((blk, hdim), lambda i: (i, 0)),
            full((hdim,)), full((hdim,)),
            full(w1.shape), full(b1.shape),
            full(w2.shape), full(b2.shape),
            full(w3.shape), full(b3.shape),
        ],
        out_specs=pl.BlockSpec((blk, ncls), lambda i: (i, 0)),
        out_shape=jax.ShapeDtypeStruct((bsz, ncls), jnp.float32),
    )(h, ln_g, ln_b, w1, b1, w2, b2, w3, b3)


def kernel(x, tables, ln_g, ln_b, W1, b1, W2, b2, W3, b3):
    bsz, f = x.shape
    _, v, d = tables.shape
    n_total = bsz * f
    idx = x + (jnp.arange(f, dtype=jnp.int32) * v)[None, :]
    idx2d = idx.reshape(_NW, n_total // (_NW * _CH), _CH)
    table_flat = tables.reshape(f * v, d)
    rows = _sc_gather(table_flat, idx2d, n_total, d)
    h = rows.reshape(bsz, f * d)
    return _tc_mlp(h, ln_g, ln_b, W1, b1, W2, b2, W3, b3, blk=512)


# speedup vs baseline: 13.4994x; 13.4994x over previous
import functools

import jax
import jax.numpy as jnp
from jax import lax
from jax.experimental import pallas as pl
from jax.experimental.pallas import tpu as pltpu
from jax.experimental.pallas import tpu_sc as plsc

_NC = 2
_NS = 16
_NW = _NC * _NS
_F = 26
_D = 16
_HP = 512
_CB = 16
_NCHUNK = 8
_LPC = _F * _CB


def _sc_gather(t3, idx3, bsz, v):
    mesh = plsc.VectorSubcoreMesh(core_axis_name="c", subcore_axis_name="s")

    @functools.partial(
        pl.kernel,
        mesh=mesh,
        out_type=jax.ShapeDtypeStruct((bsz * _HP,), jnp.float32),
        scratch_types=(
            [pltpu.VMEM((_LPC,), jnp.int32),
             pltpu.VMEM((_CB * _HP,), jnp.float32)]
            + [pltpu.VMEM((_D, 128), jnp.float32) for _ in range(_CB)]
            + [pltpu.SemaphoreType.DMA for _ in range(_CB)]
        ),
        compiler_params=pltpu.CompilerParams(
            needs_layout_passes=False, disable_bounds_checks=True),
    )
    def gather_k(t3_hbm, idx_hbm, out_hbm, idx_vv, stage_v, *rest):
        bufs = rest[:_CB]
        sems = rest[_CB:]
        wid = lax.axis_index("s") * _NC + lax.axis_index("c")
        rows16 = lax.iota(jnp.int32, 16)

        def fire(vscal, f, lane):
            c128 = pl.multiple_of((vscal >> 7) << 7, 128)
            pltpu.async_copy(
                t3_hbm.at[f, pl.ds(0, _D), pl.ds(c128, 128)],
                bufs[lane], sems[lane])

        def drain_extract(vscal, f, lane):
            pltpu.make_async_copy(
                t3_hbm.at[0, pl.ds(0, _D), pl.ds(0, 128)],
                bufs[lane], sems[lane]).wait()
            vals = plsc.load_gather(
                bufs[lane],
                [rows16, jnp.full((16,), vscal & 127, jnp.int32)])
            off = pl.multiple_of(lane * _HP + f * _D, 16)
            stage_v[pl.ds(off, _D)] = vals

        for c in range(_NCHUNK):
            pltpu.sync_copy(idx_hbm.at[wid, c], idx_vv)
            v0 = idx_vv[pl.ds(0, _CB)]
            for lane in range(_CB):
                fire(v0[lane], 0, lane)

            def group(g, vprev):
                vcur = idx_vv[pl.ds(g * _CB, _CB)]
                for lane in range(_CB):
                    drain_extract(vprev[lane], g - 1, lane)
                    fire(vcur[lane], g, lane)
                return vcur

            vlast = lax.fori_loop(1, _F, group, v0)
            for lane in range(_CB):
                drain_extract(vlast[lane], _F - 1, lane)
            pltpu.sync_copy(
                stage_v,
                out_hbm.at[pl.ds(wid * (128 * _HP) + c * (_CB * _HP),
                                 _CB * _HP)])

    return gather_k(t3, idx3)


def _mlp_body(h_ref, g_ref, b_ref, w1_ref, b1_ref, w2_ref, b2_ref,
              w3t_ref, b3_ref, o_ref):
    h = h_ref[...][:, : _F * _D]
    mu = jnp.mean(h, axis=-1, keepdims=True)
    cc = h - mu
    var = jnp.mean(cc * cc, axis=-1, keepdims=True)
    hn = cc * lax.rsqrt(var + 1e-5) * g_ref[...] + b_ref[...]
    a = jnp.dot(hn, w1_ref[...], preferred_element_type=jnp.float32)
    a = jnp.maximum(a + b1_ref[...], 0.0)
    a = jnp.dot(a, w2_ref[...], preferred_element_type=jnp.float32)
    a = jnp.maximum(a + b2_ref[...], 0.0)
    o = lax.dot_general(a, w3t_ref[...], (((1,), (1,)), ((), ())),
                        preferred_element_type=jnp.float32)
    o_ref[...] = o + b3_ref[...]


def _tc_mlp(h, ln_g, ln_b, w1, b1, w2, b2, w3t, b3, blk):
    bsz = h.shape[0]
    ncls = w3t.shape[0]
    grid = bsz // blk
    full = lambda shape: pl.BlockSpec(shape, lambda i: (0,) * len(shape))
    return pl.pallas_call(
        _mlp_body,
        grid=(grid,),
        in_specs=[
            pl.BlockSpec((blk, _HP), lambda i: (i, 0)),
            full((_F * _D,)), full((_F * _D,)),
            full(w1.shape), full(b1.shape),
            full(w2.shape), full(b2.shape),
            full(w3t.shape), full(b3.shape),
        ],
        out_specs=pl.BlockSpec((blk, ncls), lambda i: (i, 0)),
        out_shape=jax.ShapeDtypeStruct((bsz, ncls), jnp.float32),
    )(h, ln_g, ln_b, w1, b1, w2, b2, w3t, b3)


def kernel(x, tables, ln_g, ln_b, W1, b1, W2, b2, W3, b3):
    bsz, f = x.shape
    _, v, d = tables.shape
    t3 = jnp.transpose(tables, (0, 2, 1))
    w3t = W3.T
    idx3 = (x.reshape(_NW, _NCHUNK, _CB, f)
            .transpose(0, 1, 3, 2)
            .reshape(_NW, _NCHUNK, _LPC))
    hflat = _sc_gather(t3, idx3, bsz, v)
    h = hflat.reshape(bsz, _HP)
    return _tc_mlp(h, ln_g, ln_b, W1, b1, W2, b2, w3t, b3, blk=512)
